# baseline (speedup 1.0000x reference)
import functools

import jax
import jax.numpy as jnp
from jax import lax
from jax.experimental import pallas as pl
from jax.experimental.pallas import tpu as pltpu
from jax.experimental.pallas import tpu_sc as plsc

B = 4096
NC = 2
NS = 16
NW = NC * NS
BPW = B // NW
L = 16
CH = BPW // L


def _body(idx_hbm, betas_hbm, go_hbm, bp_hbm, tr_hbm,
          out_b, out_go, out_bp, out_tr,
          idx_v, b_rows, go_rows, bp_rows, tr_rows,
          sem_b, sem_g, sem_p, sem_t, osem):
    wid = lax.axis_index("s") * NC + lax.axis_index("c")
    base = wid * BPW

    pltpu.sync_copy(idx_hbm.at[pl.ds(base, BPW)], idx_v)

    def chunk(c, _):
        iv = idx_v[pl.ds(c * L, L)]
        for l in range(L):
            b = c * L + l
            r = iv[l]
            pltpu.async_copy(go_hbm.at[pl.ds(r, 1)],
                             go_rows.at[pl.ds(b, 1)], sem_g)
            pltpu.async_copy(bp_hbm.at[pl.ds(r, 1)],
                             bp_rows.at[pl.ds(b, 1)], sem_p)
            pltpu.async_copy(tr_hbm.at[pl.ds(r, 1)],
                             tr_rows.at[pl.ds(b, 1)], sem_t)
            pltpu.async_copy(betas_hbm,
                             b_rows.at[pl.ds(b, 1)], sem_b)
        pltpu.make_async_copy(go_hbm.at[pl.ds(0, L)],
                              go_rows.at[pl.ds(c * L, L)], sem_g).wait()
        pltpu.make_async_copy(bp_hbm.at[pl.ds(0, L)],
                              bp_rows.at[pl.ds(c * L, L)], sem_p).wait()
        pltpu.make_async_copy(tr_hbm.at[pl.ds(0, L)],
                              tr_rows.at[pl.ds(c * L, L)], sem_t).wait()
        pltpu.make_async_copy(betas_hbm.at[pl.ds(0, 1)],
                              b_rows.at[pl.ds(c * L, L)], sem_b).wait()
        return _

    lax.fori_loop(0, CH, chunk, None)

    ocps = [
        pltpu.async_copy(go_rows, out_go.at[pl.ds(base, BPW)], osem),
        pltpu.async_copy(tr_rows, out_tr.at[pl.ds(base, BPW)], osem),
        pltpu.async_copy(bp_rows, out_bp.at[pl.ds(base, BPW)], osem),
        pltpu.async_copy(b_rows, out_b.at[pl.ds(base, BPW)], osem),
    ]
    for cp in ocps:
        cp.wait()


def kernel(idx, betas, global_orient, body_pose, transl):
    idx = idx.astype(jnp.int32)
    db = betas.shape[1]
    dg = global_orient.shape[1]
    dp = body_pose.shape[1]
    dt = transl.shape[1]
    mesh = plsc.VectorSubcoreMesh(core_axis_name="c", subcore_axis_name="s")
    run = functools.partial(
        pl.kernel,
        mesh=mesh,
        out_type=[
            jax.ShapeDtypeStruct((B, db), jnp.float32),
            jax.ShapeDtypeStruct((B, dg), jnp.float32),
            jax.ShapeDtypeStruct((B, dp), jnp.float32),
            jax.ShapeDtypeStruct((B, dt), jnp.float32),
        ],
        scratch_types=[
            pltpu.VMEM((BPW,), jnp.int32),
            pltpu.VMEM((BPW, db), jnp.float32),
            pltpu.VMEM((BPW, dg), jnp.float32),
            pltpu.VMEM((BPW, dp), jnp.float32),
            pltpu.VMEM((BPW, dt), jnp.float32),
            pltpu.SemaphoreType.DMA,
            pltpu.SemaphoreType.DMA,
            pltpu.SemaphoreType.DMA,
            pltpu.SemaphoreType.DMA,
            pltpu.SemaphoreType.DMA,
        ],
    )(_body)
    ob, ogo, obp, otr = run(idx, betas, global_orient, body_pose, transl)
    return (ob, ogo, obp, otr)


# speedup vs baseline: 2.3468x; 2.3468x over previous
import functools

import jax
import jax.numpy as jnp
from jax import lax
from jax.experimental import pallas as pl
from jax.experimental.pallas import tpu as pltpu
from jax.experimental.pallas import tpu_sc as plsc

B = 4096
NC = 2
NS = 16
NW = NC * NS
BPW = B // NW
L = 16
CH = BPW // L
DB = 10


def _body(idx_hbm, betas_hbm, go_hbm, bp_hbm, tr_hbm,
          out_b, out_go, out_bp, out_tr,
          idx_v, bet_v, b_rows, go_rows, bp_rows, tr_rows,
          sem_g, sem_p, sem_t, osem):
    wid = lax.axis_index("s") * NC + lax.axis_index("c")
    base = wid * BPW

    pltpu.sync_copy(idx_hbm.at[pl.ds(base, BPW)], idx_v)
    pltpu.sync_copy(betas_hbm, bet_v)

    def chunk(c, _):
        iv = idx_v[pl.ds(c * L, L)]
        for l in range(L):
            b = c * L + l
            r = iv[l]
            pltpu.async_copy(go_hbm.at[pl.ds(r, 1)],
                             go_rows.at[pl.ds(b, 1)], sem_g)
            pltpu.async_copy(bp_hbm.at[pl.ds(r, 1)],
                             bp_rows.at[pl.ds(b, 1)], sem_p)
            pltpu.async_copy(tr_hbm.at[pl.ds(r, 1)],
                             tr_rows.at[pl.ds(b, 1)], sem_t)
        return _

    lax.fori_loop(0, CH, chunk, None)

    iota = lax.iota(jnp.int32, L)
    for m in range(5):
        lane = iota + 16 * m
        sel = lane
        for t in (10, 20, 30, 40, 50, 60, 70):
            sel = jnp.where(lane >= t, lane - t, sel)
        vm = plsc.load_gather(bet_v, [sel])
        for r in range(BPW * DB // 80):
            b_rows[pl.ds(80 * r + 16 * m, L)] = vm

    pltpu.make_async_copy(go_hbm.at[pl.ds(0, BPW)], go_rows, sem_g).wait()
    pltpu.make_async_copy(bp_hbm.at[pl.ds(0, BPW)], bp_rows, sem_p).wait()
    pltpu.make_async_copy(tr_hbm.at[pl.ds(0, BPW)], tr_rows, sem_t).wait()

    ocps = [
        pltpu.async_copy(go_rows, out_go.at[pl.ds(base, BPW)], osem),
        pltpu.async_copy(tr_rows, out_tr.at[pl.ds(base, BPW)], osem),
        pltpu.async_copy(bp_rows, out_bp.at[pl.ds(base, BPW)], osem),
        pltpu.async_copy(b_rows, out_b.at[pl.ds(base * DB, BPW * DB)], osem),
    ]
    for cp in ocps:
        cp.wait()


def kernel(idx, betas, global_orient, body_pose, transl):
    idx = idx.astype(jnp.int32)
    dg = global_orient.shape[1]
    dp = body_pose.shape[1]
    dt = transl.shape[1]
    bet_f = betas.reshape(-1)
    mesh = plsc.VectorSubcoreMesh(core_axis_name="c", subcore_axis_name="s")
    run = functools.partial(
        pl.kernel,
        mesh=mesh,
        compiler_params=pltpu.CompilerParams(needs_layout_passes=False),
        out_type=[
            jax.ShapeDtypeStruct((B * DB,), jnp.float32),
            jax.ShapeDtypeStruct((B, dg), jnp.float32),
            jax.ShapeDtypeStruct((B, dp), jnp.float32),
            jax.ShapeDtypeStruct((B, dt), jnp.float32),
        ],
        scratch_types=[
            pltpu.VMEM((BPW,), jnp.int32),
            pltpu.VMEM((DB,), jnp.float32),
            pltpu.VMEM((BPW * DB,), jnp.float32),
            pltpu.VMEM((BPW, dg), jnp.float32),
            pltpu.VMEM((BPW, dp), jnp.float32),
            pltpu.VMEM((BPW, dt), jnp.float32),
            pltpu.SemaphoreType.DMA,
            pltpu.SemaphoreType.DMA,
            pltpu.SemaphoreType.DMA,
            pltpu.SemaphoreType.DMA,
        ],
    )(_body)
    ob, ogo, obp, otr = run(idx, bet_f, global_orient, body_pose, transl)
    return (ob.reshape(B, DB), ogo, obp, otr)
